# baseline (speedup 1.0000x reference)
import functools
import jax
import jax.numpy as jnp
import numpy as np
from jax import lax
from jax.experimental import pallas as pl
from jax.experimental.pallas import tpu as pltpu, tpu_sc as plsc

B, L, D = 4096, 50, 64
N = B * L
NC, NS, LANES = 2, 16, 16
NW = NC * NS
TOK = N // NW
CHUNK = 128
NCH = TOK // CHUNK
VPT = D // LANES

_mesh = plsc.VectorSubcoreMesh(core_axis_name="c", subcore_axis_name="s")


@functools.partial(
    pl.kernel,
    out_type=jax.ShapeDtypeStruct((N, D), jnp.float32),
    mesh=_mesh,
    compiler_params=pltpu.CompilerParams(use_tc_tiling_on_sc=False),
    scratch_types=[
        pltpu.VMEM((CHUNK,), jnp.int32),
        pltpu.VMEM((CHUNK,), jnp.int32),
        pltpu.VMEM((CHUNK,), jnp.int32),
        pltpu.VMEM((CHUNK,), jnp.int32),
        pltpu.VMEM((CHUNK,), jnp.int32),
        pltpu.VMEM((CHUNK,), jnp.int32),
        pltpu.VMEM((CHUNK, D), jnp.float32),
        pltpu.VMEM((CHUNK, D), jnp.float32),
        pltpu.VMEM((CHUNK, D), jnp.float32),
        pltpu.VMEM((CHUNK, D), jnp.float32),
        pltpu.VMEM((CHUNK, D), jnp.float32),
        pltpu.VMEM((CHUNK, D), jnp.float32),
        pltpu.SemaphoreType.DMA,
    ],
)
def _emb_sum(ip, ic, icol, isz, ig, Wp, Wc, Wcol, Ws, Wg, ctab, out,
             vp, vc, vcol, vsz, vg, vcode, r0, r1, r2, r3, r4, r5, sem):
    wid = lax.axis_index("s") * NC + lax.axis_index("c")
    idx_refs = (vp, vc, vcol, vsz, vg)
    tab_refs = (Wp, Wc, Wcol, Ws, Wg)
    row_refs = (r0, r1, r2, r3, r4)

    def chunk_body(c, _):
        base = wid * TOK + c * CHUNK
        for iref, vref in zip((ip, ic, icol, isz, ig), idx_refs):
            pltpu.sync_copy(iref.at[pl.ds(base, CHUNK)], vref)
        for g in range(CHUNK // LANES):
            sl = pl.ds(g * LANES, LANES)
            code = jnp.where(vp[sl] == 0, 1, 0)
            for t, vref in enumerate(idx_refs[1:], start=1):
                code = code + jnp.where(vref[sl] == 0, 1 << t, 0)
            vcode[sl] = code
        copies = [
            pltpu.async_copy(tab.at[vref], rref, sem)
            for tab, vref, rref in zip(tab_refs, idx_refs, row_refs)
        ]
        copies.append(pltpu.async_copy(ctab.at[vcode], r5, sem))
        for cp in copies:
            cp.wait()
        def add_body(i, _):
            for d in range(VPT):
                sl = pl.ds(d * LANES, LANES)
                s = r0[i, sl] + r1[i, sl] + r2[i, sl]
                s = s + r3[i, sl] + r4[i, sl] + r5[i, sl]
                r0[i, sl] = s
            return 0
        lax.fori_loop(0, CHUNK, add_body, 0, unroll=2)
        pltpu.sync_copy(r0, out.at[pl.ds(base, CHUNK)])
        return 0

    lax.fori_loop(0, NCH, chunk_body, 0)


def kernel(product, customer, color, size, group,
           W_product, W_customer, W_color, W_size, W_group):
    bits = (np.arange(32)[:, None] >> np.arange(5)[None, :]) & 1
    w0 = jnp.stack([W_product[0], W_customer[0], W_color[0],
                    W_size[0], W_group[0]])
    ctab = -(jnp.asarray(bits, jnp.float32) @ w0)
    out = _emb_sum(product.reshape(-1), customer.reshape(-1),
                   color.reshape(-1), size.reshape(-1), group.reshape(-1),
                   W_product, W_customer, W_color, W_size, W_group, ctab)
    return out.reshape(B, L, D)
